# baseline (speedup 1.0000x reference)
import jax
import jax.numpy as jnp
from jax.experimental import pallas as pl
from jax.experimental.pallas import tpu as pltpu

_NS = 4


def _stream(v_hbm, buf, sems, nb, ib, make_src):
    def cp(slot, blk):
        return pltpu.make_async_copy(make_src(blk), buf.at[slot], sems.at[slot])

    @pl.when(ib == 0)
    def _():
        for s in range(min(_NS, nb)):
            cp(s, s).start()

    @pl.when(jnp.logical_and(ib > 0, ib + _NS - 1 < nb))
    def _():
        cp((ib + _NS - 1) % _NS, ib + _NS - 1).start()

    slot = ib % _NS
    cp(slot, ib).wait()
    return slot


def _pass1(v3, s0cc, s0b, n, h):
    nc = v3.shape[1]

    def body(v_hbm, scc_ref, sb_ref, row_ref, g_ref, buf, sems, sems2):
        c = pl.program_id(0)
        nh = n // 2

        def cp(slot, blk):
            return pltpu.make_async_copy(
                v_hbm.at[pl.ds(0, nh), blk, :],
                buf.at[slot, pl.ds(0, nh)], sems.at[slot])

        def cp2(slot, blk):
            return pltpu.make_async_copy(
                v_hbm.at[pl.ds(nh, nh), blk, :],
                buf.at[slot, pl.ds(nh, nh)], sems2.at[slot])

        @pl.when(c == 0)
        def _():
            for s in range(min(_NS, nc)):
                cp(s, s).start()
                cp2(s, s).start()

        @pl.when(jnp.logical_and(c > 0, c + _NS - 1 < nc))
        def _():
            cp((c + _NS - 1) % _NS, c + _NS - 1).start()
            cp2((c + _NS - 1) % _NS, c + _NS - 1).start()

        slot = c % _NS
        cp(slot, c).wait()
        cp2(slot, c).wait()
        a = buf[slot].astype(jnp.bfloat16)
        rt = jnp.dot(a, scc_ref[...], preferred_element_type=jnp.float32)
        g_ref[...] = jax.lax.dot_general(
            a, sb_ref[...], dimension_numbers=(((0,), (0,)), ((), ())),
            preferred_element_type=jnp.float32)

        @pl.when(c == 0)
        def _():
            row_ref[...] = rt

        @pl.when(c != 0)
        def _():
            row_ref[...] = row_ref[...] + rt

    return pl.pallas_call(
        body,
        grid=(nc,),
        in_specs=[
            pl.BlockSpec(memory_space=pltpu.MemorySpace.HBM),
            pl.BlockSpec((128, h), lambda c: (c, 0)),
            pl.BlockSpec((n, h), lambda c: (0, 0)),
        ],
        out_specs=[
            pl.BlockSpec((n, h), lambda c: (0, 0)),
            pl.BlockSpec((128, h), lambda c: (c, 0)),
        ],
        out_shape=[
            jax.ShapeDtypeStruct((n, h), jnp.float32),
            jax.ShapeDtypeStruct((nc * 128, h), jnp.float32),
        ],
        scratch_shapes=[
            pltpu.VMEM((_NS, n, 128), jnp.float32),
            pltpu.SemaphoreType.DMA((_NS,)),
            pltpu.SemaphoreType.DMA((_NS,)),
        ],
    )(v3, s0cc, s0b)


def _pass2(v3, s1cc, s1b, bs, n, h, ncc):
    nc = v3.shape[1]

    def body(v_hbm, scc_ref, sb_ref, row_ref, g_ref, bufa, sema, bufb, semb):
        c = pl.program_id(0)
        slot = _stream(v_hbm, bufa, sema, nc, c,
                       lambda blk: v_hbm.at[pl.ds(0, bs), blk, :])

        def cpb(s, blk):
            return pltpu.make_async_copy(
                v_hbm.at[pl.ds(0, n), blk, :], bufb.at[s], semb.at[s])

        @pl.when(c == 0)
        def _():
            for s in range(min(_NS, ncc)):
                cpb(s, s).start()

        @pl.when(jnp.logical_and(c > 0, c + _NS - 1 < ncc))
        def _():
            cpb((c + _NS - 1) % _NS, c + _NS - 1).start()

        a = bufa[slot].astype(jnp.bfloat16)
        rt = jnp.dot(a, scc_ref[...], preferred_element_type=jnp.float32)

        @pl.when(c == 0)
        def _():
            row_ref[...] = rt

        @pl.when(c != 0)
        def _():
            row_ref[...] = row_ref[...] + rt

        @pl.when(c < ncc)
        def _():
            sb = c % _NS
            cpb(sb, c).wait()
            b = bufb[sb].astype(jnp.bfloat16)
            g_ref[...] = jax.lax.dot_general(
                b, sb_ref[...], dimension_numbers=(((0,), (0,)), ((), ())),
                preferred_element_type=jnp.float32)

    return pl.pallas_call(
        body,
        grid=(nc,),
        in_specs=[
            pl.BlockSpec(memory_space=pltpu.MemorySpace.HBM),
            pl.BlockSpec((128, h), lambda c: (c, 0)),
            pl.BlockSpec((n, h), lambda c: (0, 0)),
        ],
        out_specs=[
            pl.BlockSpec((bs, h), lambda c: (0, 0)),
            pl.BlockSpec((128, h), lambda c: (jnp.minimum(c, ncc - 1), 0)),
        ],
        out_shape=[
            jax.ShapeDtypeStruct((bs, h), jnp.float32),
            jax.ShapeDtypeStruct((ncc * 128, h), jnp.float32),
        ],
        scratch_shapes=[
            pltpu.VMEM((_NS, bs, 128), jnp.float32),
            pltpu.SemaphoreType.DMA((_NS,)),
            pltpu.VMEM((_NS, n, 128), jnp.float32),
            pltpu.SemaphoreType.DMA((_NS,)),
        ],
    )(v3, s1cc, s1b)


def _leaky(x):
    return jnp.where(x >= 0, x, 0.01 * x)


def _heads(ge, x_top, tweets, pe_w0, pe_b0, pe_wo, pe_bo,
           w1a, w1b, w1c, bc_b1, bc_w2, bc_b2):
    bs = tweets.shape[0]

    def body(ge_ref, xp_ref, tw_ref, pw0_ref, pb0_ref, pwo_ref, pbo_ref,
             w1a_ref, w1b_ref, w1c_ref, b1_ref, w2_ref, b2_ref, out_ref):
        hp = jnp.dot(xp_ref[...], pw0_ref[...], preferred_element_type=jnp.float32)
        hp = _leaky(hp + pb0_ref[...])
        prop = jnp.dot(hp, pwo_ref[...], preferred_element_type=jnp.float32) + pbo_ref[...]
        hid = (jnp.dot(ge_ref[...], w1a_ref[...], preferred_element_type=jnp.float32)
               + jnp.dot(prop, w1b_ref[...], preferred_element_type=jnp.float32)
               + jnp.dot(tw_ref[...], w1c_ref[...], preferred_element_type=jnp.float32)
               + b1_ref[...])
        hid = _leaky(hid)
        logits = _leaky(jnp.dot(hid, w2_ref[...], preferred_element_type=jnp.float32)
                        + b2_ref[...])
        m = jnp.max(logits, axis=-1, keepdims=True)
        e = jnp.exp(logits - m)
        out_ref[...] = e / jnp.sum(e, axis=-1, keepdims=True)

    return pl.pallas_call(
        body,
        out_shape=jax.ShapeDtypeStruct((bs, 2), jnp.float32),
    )(ge, x_top, tweets, pe_w0, pe_b0, pe_wo, pe_bo,
      w1a, w1b, w1c, bc_b1, bc_w2, bc_b2)


def _chunk_weighted(s, w, n, h):
    r = w.shape[0]
    sr = s.reshape(n // 128, 1, 128, h) * w[None, :, None, None]
    return sr.reshape(n * r // 128 * 128, h).astype(jnp.bfloat16)


def _pair_combine(g, w, h):
    r = w.shape[0]
    return (g.reshape(-1, r, 128, h) * w[None, :, None, None]).sum(axis=1).reshape(-1, h)


def kernel(x_feature, adj_matrix, des, tweets, batch_size,
           rel_w, gc_w0, gc_b0, gc_w1, gc_b1,
           pe_w0, pe_b0, pe_wo, pe_bo,
           bc_w1, bc_b1, bc_w2, bc_b2):
    n, f = x_feature.shape
    r = adj_matrix.shape[2]
    h = gc_w0.shape[1]
    bs, t = tweets.shape

    v3 = (adj_matrix.reshape(n, n // 128, 128, r)
          .transpose(0, 1, 3, 2)
          .reshape(n, n * r // 128, 128))
    w = rel_w[:, 0]

    s0 = jnp.dot(x_feature, gc_w0)
    s0cc = _chunk_weighted(s0, w, n, h)
    row1, g1 = _pass1(v3, s0cc, s0.astype(jnp.bfloat16), n, h)
    col1 = _pair_combine(g1, w, h)
    out0 = row1 + col1 + gc_b0[None, :]

    s1 = jnp.dot(out0, gc_w1)
    s1cc = _chunk_weighted(s1, w, n, h)
    row2, g2 = _pass2(v3, s1cc, s1.astype(jnp.bfloat16), bs, n, h,
                      ncc=bs * r // 128)
    col2 = _pair_combine(g2, w, h)
    out1_top = row2 + col2 + gc_b1[None, :]

    graph_emb = 0.5 * (out0[:bs] + out1_top)

    x_top = x_feature[:bs]
    return _heads(graph_emb, x_top, tweets,
                  pe_w0, pe_b0.reshape(1, h), pe_wo, pe_bo.reshape(1, h),
                  bc_w1[:h], bc_w1[h:2 * h], bc_w1[2 * h:],
                  bc_b1.reshape(1, h), bc_w2, bc_b2.reshape(1, 2))


# speedup vs baseline: 1.0053x; 1.0053x over previous
import jax
import jax.numpy as jnp
from jax.experimental import pallas as pl
from jax.experimental.pallas import tpu as pltpu

_NS = 4


def _stream(v_hbm, buf, sems, nb, ib, make_src):
    def cp(slot, blk):
        return pltpu.make_async_copy(make_src(blk), buf.at[slot], sems.at[slot])

    @pl.when(ib == 0)
    def _():
        for s in range(min(_NS, nb)):
            cp(s, s).start()

    @pl.when(jnp.logical_and(ib > 0, ib + _NS - 1 < nb))
    def _():
        cp((ib + _NS - 1) % _NS, ib + _NS - 1).start()

    slot = ib % _NS
    cp(slot, ib).wait()
    return slot


def _pass1(v3, s0cc, s0b, n, h):
    nc = v3.shape[1]

    def body(v_hbm, scc_ref, sb_ref, row_ref, g_ref, buf, sems):
        c = pl.program_id(0)
        slot = _stream(v_hbm, buf, sems, nc, c,
                       lambda blk: v_hbm.at[pl.ds(0, n), blk, :])
        a = buf[slot].astype(jnp.bfloat16)
        rt = jnp.dot(a, scc_ref[...], preferred_element_type=jnp.float32)
        g_ref[...] = jax.lax.dot_general(
            a, sb_ref[...], dimension_numbers=(((0,), (0,)), ((), ())),
            preferred_element_type=jnp.float32)

        @pl.when(c == 0)
        def _():
            row_ref[...] = rt

        @pl.when(c != 0)
        def _():
            row_ref[...] = row_ref[...] + rt

    return pl.pallas_call(
        body,
        grid=(nc,),
        in_specs=[
            pl.BlockSpec(memory_space=pltpu.MemorySpace.HBM),
            pl.BlockSpec((128, h), lambda c: (c, 0)),
            pl.BlockSpec((n, h), lambda c: (0, 0)),
        ],
        out_specs=[
            pl.BlockSpec((n, h), lambda c: (0, 0)),
            pl.BlockSpec((128, h), lambda c: (c, 0)),
        ],
        out_shape=[
            jax.ShapeDtypeStruct((n, h), jnp.float32),
            jax.ShapeDtypeStruct((nc * 128, h), jnp.float32),
        ],
        scratch_shapes=[
            pltpu.VMEM((_NS, n, 128), jnp.float32),
            pltpu.SemaphoreType.DMA((_NS,)),
        ],
    )(v3, s0cc, s0b)


def _pass2(v3, s1cc, s1b, bs, n, h, ncc):
    nc = v3.shape[1]

    def body(v_hbm, scc_ref, sb_ref, row_ref, g_ref, bufa, sema, bufb, semb):
        c = pl.program_id(0)
        slot = _stream(v_hbm, bufa, sema, nc, c,
                       lambda blk: v_hbm.at[pl.ds(0, bs), blk, :])

        def cpb(s, blk):
            return pltpu.make_async_copy(
                v_hbm.at[pl.ds(0, n), blk, :], bufb.at[s], semb.at[s])

        @pl.when(c == 0)
        def _():
            for s in range(min(_NS, ncc)):
                cpb(s, s).start()

        @pl.when(jnp.logical_and(c > 0, c + _NS - 1 < ncc))
        def _():
            cpb((c + _NS - 1) % _NS, c + _NS - 1).start()

        a = bufa[slot].astype(jnp.bfloat16)
        rt = jnp.dot(a, scc_ref[...], preferred_element_type=jnp.float32)

        @pl.when(c == 0)
        def _():
            row_ref[...] = rt

        @pl.when(c != 0)
        def _():
            row_ref[...] = row_ref[...] + rt

        @pl.when(c < ncc)
        def _():
            sb = c % _NS
            cpb(sb, c).wait()
            b = bufb[sb].astype(jnp.bfloat16)
            g_ref[...] = jax.lax.dot_general(
                b, sb_ref[...], dimension_numbers=(((0,), (0,)), ((), ())),
                preferred_element_type=jnp.float32)

    return pl.pallas_call(
        body,
        grid=(nc,),
        in_specs=[
            pl.BlockSpec(memory_space=pltpu.MemorySpace.HBM),
            pl.BlockSpec((128, h), lambda c: (c, 0)),
            pl.BlockSpec((n, h), lambda c: (0, 0)),
        ],
        out_specs=[
            pl.BlockSpec((bs, h), lambda c: (0, 0)),
            pl.BlockSpec((128, h), lambda c: (jnp.minimum(c, ncc - 1), 0)),
        ],
        out_shape=[
            jax.ShapeDtypeStruct((bs, h), jnp.float32),
            jax.ShapeDtypeStruct((ncc * 128, h), jnp.float32),
        ],
        scratch_shapes=[
            pltpu.VMEM((_NS, bs, 128), jnp.float32),
            pltpu.SemaphoreType.DMA((_NS,)),
            pltpu.VMEM((_NS, n, 128), jnp.float32),
            pltpu.SemaphoreType.DMA((_NS,)),
        ],
    )(v3, s1cc, s1b)


def _leaky(x):
    return jnp.where(x >= 0, x, 0.01 * x)


def _heads(ge, x_top, tweets, pe_w0, pe_b0, pe_wo, pe_bo,
           w1a, w1b, w1c, bc_b1, bc_w2, bc_b2):
    bs = tweets.shape[0]

    def body(ge_ref, xp_ref, tw_ref, pw0_ref, pb0_ref, pwo_ref, pbo_ref,
             w1a_ref, w1b_ref, w1c_ref, b1_ref, w2_ref, b2_ref, out_ref):
        hp = jnp.dot(xp_ref[...], pw0_ref[...], preferred_element_type=jnp.float32)
        hp = _leaky(hp + pb0_ref[...])
        prop = jnp.dot(hp, pwo_ref[...], preferred_element_type=jnp.float32) + pbo_ref[...]
        hid = (jnp.dot(ge_ref[...], w1a_ref[...], preferred_element_type=jnp.float32)
               + jnp.dot(prop, w1b_ref[...], preferred_element_type=jnp.float32)
               + jnp.dot(tw_ref[...], w1c_ref[...], preferred_element_type=jnp.float32)
               + b1_ref[...])
        hid = _leaky(hid)
        logits = _leaky(jnp.dot(hid, w2_ref[...], preferred_element_type=jnp.float32)
                        + b2_ref[...])
        m = jnp.max(logits, axis=-1, keepdims=True)
        e = jnp.exp(logits - m)
        out_ref[...] = e / jnp.sum(e, axis=-1, keepdims=True)

    return pl.pallas_call(
        body,
        out_shape=jax.ShapeDtypeStruct((bs, 2), jnp.float32),
    )(ge, x_top, tweets, pe_w0, pe_b0, pe_wo, pe_bo,
      w1a, w1b, w1c, bc_b1, bc_w2, bc_b2)


def _chunk_weighted(s, w, n, h):
    r = w.shape[0]
    sr = s.reshape(n // 128, 1, 128, h) * w[None, :, None, None]
    return sr.reshape(n * r // 128 * 128, h).astype(jnp.bfloat16)


def _pair_combine(g, w, h):
    r = w.shape[0]
    return (g.reshape(-1, r, 128, h) * w[None, :, None, None]).sum(axis=1).reshape(-1, h)


def kernel(x_feature, adj_matrix, des, tweets, batch_size,
           rel_w, gc_w0, gc_b0, gc_w1, gc_b1,
           pe_w0, pe_b0, pe_wo, pe_bo,
           bc_w1, bc_b1, bc_w2, bc_b2):
    n, f = x_feature.shape
    r = adj_matrix.shape[2]
    h = gc_w0.shape[1]
    bs, t = tweets.shape

    v3 = (adj_matrix.reshape(n, n // 128, 128, r)
          .transpose(0, 1, 3, 2)
          .reshape(n, n * r // 128, 128))
    w = rel_w[:, 0]

    s0 = jnp.dot(x_feature, gc_w0)
    s0cc = _chunk_weighted(s0, w, n, h)
    row1, g1 = _pass1(v3, s0cc, s0.astype(jnp.bfloat16), n, h)
    col1 = _pair_combine(g1, w, h)
    out0 = row1 + col1 + gc_b0[None, :]

    s1 = jnp.dot(out0, gc_w1)
    s1cc = _chunk_weighted(s1, w, n, h)
    row2, g2 = _pass2(v3, s1cc, s1.astype(jnp.bfloat16), bs, n, h,
                      ncc=bs * r // 128)
    col2 = _pair_combine(g2, w, h)
    out1_top = row2 + col2 + gc_b1[None, :]

    graph_emb = 0.5 * (out0[:bs] + out1_top)

    x_top = x_feature[:bs]
    return _heads(graph_emb, x_top, tweets,
                  pe_w0, pe_b0.reshape(1, h), pe_wo, pe_bo.reshape(1, h),
                  bc_w1[:h], bc_w1[h:2 * h], bc_w1[2 * h:],
                  bc_b1.reshape(1, h), bc_w2, bc_b2.reshape(1, 2))
